# baseline (device time: 41878 ns/iter reference)
import jax
import jax.numpy as jnp
from jax import lax
from jax.experimental import pallas as pl
from jax.experimental.pallas import tpu as pltpu

P = 8
N_TOK = 1024
D = 256
H = 512
N_EXP = 32
E_LOCAL = N_EXP // P
CHUNK = N_TOK // P

MASKS = [1, 3, 4]
BITVAL = [1, 2, 4]
AG_ORDERS = [[0, 1, 2], [1, 2, 0], [2, 0, 1]]
AG_OFF = [0, 48, 88]
AG_LEN = [48, 40, 40]
N_AG_SEMS = 3 * 7


def _virt(r):
    return (r & 6) | ((r ^ (r >> 1)) & 1)


def kernel(x, router_W, route_idx, expert_W, shared_W):
    def body(x_ref, rw_ref, idx_ref, ew_ref, sw_ref, out_ref,
             acc_ref, gate_ref, rbuf,
             rs_send_sems, rs_recv_sems, ag_send_sems, ag_recv_sems):
        my = lax.axis_index("i")
        v = _virt(my)

        barrier = pltpu.get_barrier_semaphore()
        for j in range(1, P):
            pl.semaphore_signal(
                barrier, inc=1,
                device_id=(my ^ j,), device_id_type=pl.DeviceIdType.MESH,
            )
        pl.semaphore_wait(barrier, P - 1)

        scores = jnp.dot(x_ref[:, :], rw_ref[:, :],
                         preferred_element_type=jnp.float32)
        m = jnp.max(scores, axis=-1, keepdims=True)
        p = jnp.exp(scores - m)
        p = p / jnp.sum(p, axis=-1, keepdims=True)
        eids = lax.broadcasted_iota(jnp.int32, (N_TOK, N_EXP), 1)
        gate = jnp.sum(jnp.where(eids == idx_ref[:, :], p, 0.0),
                       axis=-1, keepdims=True)
        gate_ref[:, :] = gate

        def compute_block(c):
            row = c * CHUNK
            xb = x_ref[pl.ds(row, CHUNK), :]
            ib = idx_ref[pl.ds(row, CHUNK), :]
            gb = gate_ref[pl.ds(row, CHUNK), :]
            accb = jnp.dot(xb, sw_ref[:, :],
                           preferred_element_type=jnp.float32) * (1.0 / P)
            for ej in range(E_LOCAL):
                e = my * E_LOCAL + ej
                w = jnp.where(ib == e, gb, 0.0)
                accb = accb + jnp.dot(xb * w, ew_ref[ej],
                                      preferred_element_type=jnp.float32)
            return accb

        rs_rdmas = []
        for j in range(1, P):
            dest = my ^ j
            c = _virt(dest)
            acc_ref[pl.ds(c * CHUNK, CHUNK), :] = compute_block(c)
            rdma = pltpu.make_async_remote_copy(
                src_ref=acc_ref.at[pl.ds(c * CHUNK, CHUNK), :],
                dst_ref=rbuf.at[j - 1],
                send_sem=rs_send_sems.at[j - 1],
                recv_sem=rs_recv_sems.at[j - 1],
                device_id=(dest,),
                device_id_type=pl.DeviceIdType.MESH,
            )
            rdma.start()
            rs_rdmas.append(rdma)

        vrow = v * CHUNK
        red = compute_block(v)
        for j in range(1, P):
            rs_rdmas[j - 1].wait_recv()
            red = red + rbuf[j - 1]
        out_ref[pl.ds(vrow, CHUNK), :] = red

        sem_i = 0
        pending = []
        spans = [[0], [0], [0]]
        recv_waits = [[], [], []]
        for k in range(3):
            for q in range(3):
                for d in recv_waits[q]:
                    d.wait_recv()
                recv_waits[q] = []
                dim = AG_ORDERS[q][k]
                partner = my ^ MASKS[dim]
                bv = BITVAL[dim]
                for s in list(spans[q]):
                    c = v ^ s
                    row = c * CHUNK + AG_OFF[q]
                    send = pltpu.make_async_remote_copy(
                        src_ref=out_ref.at[pl.ds(row, AG_LEN[q]), :],
                        dst_ref=out_ref.at[pl.ds(row, AG_LEN[q]), :],
                        send_sem=ag_send_sems.at[sem_i],
                        recv_sem=ag_recv_sems.at[sem_i],
                        device_id=(partner,),
                        device_id_type=pl.DeviceIdType.MESH,
                    )
                    send.start()
                    pending.append(send)
                    crow = (c ^ bv) * CHUNK + AG_OFF[q]
                    recv = pltpu.make_async_remote_copy(
                        src_ref=out_ref.at[pl.ds(crow, AG_LEN[q]), :],
                        dst_ref=out_ref.at[pl.ds(crow, AG_LEN[q]), :],
                        send_sem=ag_send_sems.at[sem_i],
                        recv_sem=ag_recv_sems.at[sem_i],
                        device_id=(partner,),
                        device_id_type=pl.DeviceIdType.MESH,
                    )
                    recv_waits[q].append(recv)
                    spans[q].append(s ^ bv)
                    sem_i += 1
        for q in range(3):
            for d in recv_waits[q]:
                d.wait_recv()
        for d in rs_rdmas + pending:
            d.wait_send()

    return pl.pallas_call(
        body,
        out_shape=jax.ShapeDtypeStruct((N_TOK, H), jnp.float32),
        in_specs=[
            pl.BlockSpec(memory_space=pltpu.VMEM),
            pl.BlockSpec(memory_space=pltpu.VMEM),
            pl.BlockSpec(memory_space=pltpu.VMEM),
            pl.BlockSpec(memory_space=pltpu.VMEM),
            pl.BlockSpec(memory_space=pltpu.VMEM),
        ],
        out_specs=pl.BlockSpec(memory_space=pltpu.VMEM),
        scratch_shapes=[
            pltpu.VMEM((N_TOK, H), jnp.float32),
            pltpu.VMEM((N_TOK, 1), jnp.float32),
            pltpu.VMEM((P - 1, CHUNK, H), jnp.float32),
            pltpu.SemaphoreType.DMA((P - 1,)),
            pltpu.SemaphoreType.DMA((P - 1,)),
            pltpu.SemaphoreType.DMA((N_AG_SEMS,)),
            pltpu.SemaphoreType.DMA((N_AG_SEMS,)),
        ],
        compiler_params=pltpu.CompilerParams(collective_id=0),
    )(x, router_W, route_idx, expert_W, shared_W)


# device time: 28341 ns/iter; 1.4776x vs baseline; 1.4776x over previous
import jax
import jax.numpy as jnp
from jax import lax
from jax.experimental import pallas as pl
from jax.experimental.pallas import tpu as pltpu

P = 8
N_TOK = 1024
D = 256
H = 512
N_EXP = 32
E_LOCAL = N_EXP // P
CHUNK = N_TOK // P

MASKS = [1, 3, 4]
BITVAL = [1, 2, 4]
AG_ORDERS = [[0, 1, 2], [1, 2, 0], [2, 0, 1]]
AG_OFF = [0, 48, 96]
AG_LEN = [48, 48, 32]
N_AG_SEMS = 3 * 7

RS_PAIRS = [(6, 7), (2, 3), (4, 5), (1, 0)]
RS_WAIT_ORDER = [6, 7, 2, 3, 4, 5, 1]


def _virt(r):
    return (r & 6) | ((r ^ (r >> 1)) & 1)


def kernel(x, router_W, route_idx, expert_W, shared_W):
    def body(x_ref, rw_ref, idx_ref, ew_ref, sw_ref, out_ref,
             acc_ref, accbf_ref, stage_ref, gate_ref, rbuf,
             xbf_ref, wcat_ref,
             rs_send_sems, rs_recv_sems, ag_send_sems, ag_recv_sems):
        my = lax.axis_index("i")
        v = _virt(my)

        barrier = pltpu.get_barrier_semaphore()
        for j in range(1, P):
            pl.semaphore_signal(
                barrier, inc=1,
                device_id=(my ^ j,), device_id_type=pl.DeviceIdType.MESH,
            )

        scores = jnp.dot(x_ref[:, :], rw_ref[:, :],
                         preferred_element_type=jnp.float32)
        m = jnp.max(scores, axis=-1, keepdims=True)
        p = jnp.exp(scores - m)
        p = p / jnp.sum(p, axis=-1, keepdims=True)
        eids = lax.broadcasted_iota(jnp.int32, (N_TOK, N_EXP), 1)
        gate = jnp.sum(jnp.where(eids == idx_ref[:, :], p, 0.0),
                       axis=-1, keepdims=True)
        gate_ref[:, :] = gate

        xbf_ref[:, :] = x_ref[:, :].astype(jnp.bfloat16)
        wcat_ref[pl.ds(0, E_LOCAL * D), :] = ew_ref[:, :, :].astype(
            jnp.bfloat16).reshape(E_LOCAL * D, H)
        wcat_ref[pl.ds(E_LOCAL * D, D), :] = (
            sw_ref[:, :] * (1.0 / P)).astype(jnp.bfloat16)

        def compute_rows(row, n):
            xb = xbf_ref[pl.ds(row, n), :]
            ib = idx_ref[pl.ds(row, n), :]
            gb = gate_ref[pl.ds(row, n), :]
            gxb = (x_ref[pl.ds(row, n), :] * gb).astype(jnp.bfloat16)
            zero = jnp.zeros_like(gxb)
            parts = []
            for ej in range(E_LOCAL):
                e = my * E_LOCAL + ej
                parts.append(jnp.where(ib == e, gxb, zero))
            parts.append(xb)
            xcat = jnp.concatenate(parts, axis=1)
            return jnp.dot(xcat, wcat_ref[:, :],
                           preferred_element_type=jnp.float32)

        pl.semaphore_wait(barrier, P - 1)

        rs_rdmas = {}

        def send_chunk(j):
            c = _virt(my ^ j)
            rdma = pltpu.make_async_remote_copy(
                src_ref=accbf_ref.at[pl.ds(c * CHUNK, CHUNK), :],
                dst_ref=rbuf.at[j - 1],
                send_sem=rs_send_sems.at[j - 1],
                recv_sem=rs_recv_sems.at[j - 1],
                device_id=(my ^ j,),
                device_id_type=pl.DeviceIdType.MESH,
            )
            rdma.start()
            rs_rdmas[j] = rdma

        for ja, jb in RS_PAIRS[:3]:
            pair_base = (_virt(my ^ ja) & 6) * CHUNK
            accb = compute_rows(pair_base, 2 * CHUNK)
            accbf_ref[pl.ds(pair_base, 2 * CHUNK), :] = accb.astype(
                jnp.bfloat16)
            send_chunk(ja)
            send_chunk(jb)
        c1row = (v ^ 1) * CHUNK
        accbf_ref[pl.ds(c1row, CHUNK), :] = compute_rows(
            c1row, CHUNK).astype(jnp.bfloat16)
        send_chunk(1)
        vrow = v * CHUNK
        acc_ref[pl.ds(vrow, CHUNK), :] = compute_rows(vrow, CHUNK)

        red = acc_ref[pl.ds(vrow, CHUNK), :]
        for j in RS_WAIT_ORDER:
            rs_rdmas[j].wait_recv()
            red = red + rbuf[j - 1].astype(jnp.float32)
        stage_ref[pl.ds(vrow, CHUNK), :] = red.astype(jnp.bfloat16)

        pending = []
        ag_recvs = [[None] * 7 for _ in range(3)]

        SENDS = [(0, 0, 0), (1, 0, 1), (2, 0, 2),
                 (3, 1, 1), (4, 1, 2), (5, 2, 2), (6, 3, 2)]

        def start_ag(q, slots):
            o = AG_ORDERS[q]
            bvs = [0, BITVAL[o[0]], BITVAL[o[1]], BITVAL[o[0]] ^ BITVAL[o[1]]]
            for slot, si, pos in SENDS:
                if slot not in slots:
                    continue
                dim = o[pos]
                partner = my ^ MASKS[dim]
                sem_i = q * 7 + slot
                row = (v ^ bvs[si]) * CHUNK + AG_OFF[q]
                send = pltpu.make_async_remote_copy(
                    src_ref=stage_ref.at[pl.ds(row, AG_LEN[q]), :],
                    dst_ref=stage_ref.at[pl.ds(row, AG_LEN[q]), :],
                    send_sem=ag_send_sems.at[sem_i],
                    recv_sem=ag_recv_sems.at[sem_i],
                    device_id=(partner,),
                    device_id_type=pl.DeviceIdType.MESH,
                )
                send.start()
                pending.append(send)
                crow = (v ^ bvs[si] ^ BITVAL[dim]) * CHUNK + AG_OFF[q]
                recv = pltpu.make_async_remote_copy(
                    src_ref=stage_ref.at[pl.ds(crow, AG_LEN[q]), :],
                    dst_ref=stage_ref.at[pl.ds(crow, AG_LEN[q]), :],
                    send_sem=ag_send_sems.at[sem_i],
                    recv_sem=ag_recv_sems.at[sem_i],
                    device_id=(partner,),
                    device_id_type=pl.DeviceIdType.MESH,
                )
                ag_recvs[q][slot] = recv

        for q in range(3):
            start_ag(q, (0, 1, 2))
        for q in range(3):
            ag_recvs[q][0].wait_recv()
            start_ag(q, (3, 4))
        for q in range(3):
            ag_recvs[q][1].wait_recv()
            ag_recvs[q][3].wait_recv()
            start_ag(q, (5, 6))
        def convert_regions(q, spans):
            o = AG_ORDERS[q]
            bv0, bv1, bv2 = (BITVAL[o[0]], BITVAL[o[1]], BITVAL[o[2]])
            for s in spans(bv0, bv1, bv2):
                r = (v ^ s) * CHUNK + AG_OFF[q]
                out_ref[pl.ds(r, AG_LEN[q]), :] = stage_ref[
                    pl.ds(r, AG_LEN[q]), :].astype(jnp.float32)
        for q in range(3):
            convert_regions(q, lambda a, b, c: [0, a, b, a ^ b])
        for q in range(3):
            for slot in (2, 4, 5, 6):
                ag_recvs[q][slot].wait_recv()
            convert_regions(q, lambda a, b, c: [c, a ^ c, b ^ c, a ^ b ^ c])
        for d in list(rs_rdmas.values()) + pending:
            d.wait_send()

    return pl.pallas_call(
        body,
        out_shape=jax.ShapeDtypeStruct((N_TOK, H), jnp.float32),
        in_specs=[
            pl.BlockSpec(memory_space=pltpu.VMEM),
            pl.BlockSpec(memory_space=pltpu.VMEM),
            pl.BlockSpec(memory_space=pltpu.VMEM),
            pl.BlockSpec(memory_space=pltpu.VMEM),
            pl.BlockSpec(memory_space=pltpu.VMEM),
        ],
        out_specs=pl.BlockSpec(memory_space=pltpu.VMEM),
        scratch_shapes=[
            pltpu.VMEM((N_TOK, H), jnp.float32),
            pltpu.VMEM((N_TOK, H), jnp.bfloat16),
            pltpu.VMEM((N_TOK, H), jnp.bfloat16),
            pltpu.VMEM((N_TOK, 1), jnp.float32),
            pltpu.VMEM((P - 1, CHUNK, H), jnp.bfloat16),
            pltpu.VMEM((N_TOK, D), jnp.bfloat16),
            pltpu.VMEM(((E_LOCAL + 1) * D, H), jnp.bfloat16),
            pltpu.SemaphoreType.DMA((P - 1,)),
            pltpu.SemaphoreType.DMA((P - 1,)),
            pltpu.SemaphoreType.DMA((N_AG_SEMS,)),
            pltpu.SemaphoreType.DMA((N_AG_SEMS,)),
        ],
        compiler_params=pltpu.CompilerParams(collective_id=0),
    )(x, router_W, route_idx, expert_W, shared_W)
